# baseline (speedup 1.0000x reference)
import functools

import jax
import jax.numpy as jnp
from jax import lax
from jax.experimental import pallas as pl
from jax.experimental.pallas import tpu as pltpu
from jax.experimental.pallas import tpu_sc as plsc

_N_NODES = 50000
_N_EDGES = 1600000
_N_UNARY = 8
_N_BINARY = 2

_U_ROWS = (_N_NODES * _N_UNARY) // 16
_B_ROWS = (_N_EDGES * _N_BINARY) // 128

_NC = 2
_NS = 16
_NW = _NC * _NS

_MAIN = 784
_LAST = _B_ROWS - 31 * _MAIN


def _sc_copy(u_hbm, b_hbm, uo_hbm, bo_hbm, u_buf, b_buf, sem_u, sem_b):
    wid = lax.axis_index("s") * _NC + lax.axis_index("c")
    base = wid * _MAIN

    def do_copy(rows):
        cb = pltpu.make_async_copy(
            b_hbm.at[pl.ds(base, rows)], b_buf.at[pl.ds(0, rows)], sem_b
        )
        cb.start()
        cu = pltpu.make_async_copy(
            u_hbm.at[pl.ds(base, rows)], u_buf.at[pl.ds(0, rows)], sem_u
        )
        cu.start()
        cu.wait()
        pltpu.sync_copy(
            u_buf.at[pl.ds(0, rows)], uo_hbm.at[pl.ds(base, rows)]
        )
        cb.wait()
        pltpu.sync_copy(
            b_buf.at[pl.ds(0, rows)], bo_hbm.at[pl.ds(base, rows)]
        )

    @pl.when(wid < _NW - 1)
    def _():
        do_copy(_MAIN)

    @pl.when(wid == _NW - 1)
    def _():
        do_copy(_LAST)


def kernel(unary, binary, index1, index2):
    u2 = unary.reshape(_U_ROWS, 16)
    b2 = binary.reshape(_B_ROWS, 128)
    mesh = plsc.VectorSubcoreMesh(core_axis_name="c", subcore_axis_name="s")
    run = functools.partial(
        pl.kernel,
        mesh=mesh,
        out_type=[
            jax.ShapeDtypeStruct((_U_ROWS, 16), unary.dtype),
            jax.ShapeDtypeStruct((_B_ROWS, 128), binary.dtype),
        ],
        scratch_types=[
            pltpu.VMEM((_MAIN, 16), jnp.float32),
            pltpu.VMEM((_MAIN, 128), jnp.float32),
            pltpu.SemaphoreType.DMA,
            pltpu.SemaphoreType.DMA,
        ],
        compiler_params=pltpu.CompilerParams(use_tc_tiling_on_sc=False),
    )(_sc_copy)
    uo, bo = run(u2, b2)
    return (uo.reshape(unary.shape), bo.reshape(binary.shape))
